# baseline (speedup 1.0000x reference)
import functools

import jax
import jax.numpy as jnp
from jax import lax
from jax.experimental import pallas as pl
from jax.experimental.pallas import tpu as pltpu
from jax.experimental.pallas import tpu_sc as plsc

_B = 16384
_V = 1000000
_D = 16
_NC = 2
_NS = 16
_NW = _NC * _NS
_BPW = _B // _NW
_CHUNK = 128
_NCHUNK = _BPW // _CHUNK
_G = _BPW // _D

_BC = 16384
_NBLK = -(-_V // _BC)
_PS = _NBLK * _BC

_mesh = plsc.VectorSubcoreMesh(core_axis_name="c", subcore_axis_name="s")


def _proj_body(e1_ref, e2_ref, wm_ref, b_ref, p_ref):
    k = pl.program_id(0) % 3
    e1 = e1_ref[...]
    e2 = e2_ref[...]
    wm = wm_ref[...]
    wv = jnp.where(k == 0, wm[0:1], jnp.where(k == 1, wm[1:2], wm[2:3]))
    tbl = jnp.where(k == 2, e2, e1)
    chunk = jax.lax.dot_general(
        wv, tbl, (((1,), (0,)), ((), ())),
        preferred_element_type=jnp.float32)
    chunk = chunk + jnp.where(k == 2, b_ref[0, 0], 0.0)
    p_ref[...] = chunk.reshape(_BC)


_proj = pl.pallas_call(
    _proj_body,
    grid=(3 * _NBLK,),
    in_specs=[
        pl.BlockSpec((_D, _BC), lambda s: (0, s // 3)),
        pl.BlockSpec((_D, _BC), lambda s: (0, s // 3)),
        pl.BlockSpec((3, _D), lambda s: (0, 0)),
        pl.BlockSpec(memory_space=pltpu.SMEM),
    ],
    out_specs=pl.BlockSpec((_BC,), lambda s: ((s % 3) * _NBLK + s // 3,)),
    out_shape=jax.ShapeDtypeStruct((3 * _PS,), jnp.float32),
)


@functools.partial(
    pl.kernel,
    out_type=jax.ShapeDtypeStruct((_NW, _BPW), jnp.float32),
    mesh=_mesh,
    scratch_types=[
        pltpu.VMEM((_NCHUNK, _CHUNK), jnp.int32),
        pltpu.VMEM((_NCHUNK, _CHUNK), jnp.int32),
        pltpu.VMEM((_NCHUNK, _CHUNK), jnp.int32),
        pltpu.VMEM((_BPW,), jnp.float32),
        pltpu.VMEM((_BPW,), jnp.float32),
        pltpu.VMEM((_BPW,), jnp.float32),
        pltpu.VMEM((_BPW,), jnp.float32),
        pltpu.SemaphoreType.DMA,
    ],
    compiler_params=pltpu.CompilerParams(
        needs_layout_passes=False, use_tc_tiling_on_sc=False),
)
def _gather_add_sc(f1_hbm, f2_hbm, f3_hbm, p_hbm, out_hbm,
                   idx1, idx2, idx3, g1, g2, g3, out_v, sem):
    wid = lax.axis_index("s") * _NC + lax.axis_index("c")

    pltpu.sync_copy(f1_hbm.at[wid], idx1)
    pltpu.sync_copy(f2_hbm.at[wid], idx2)
    pltpu.sync_copy(f3_hbm.at[wid], idx3)

    for off, idx in ((_PS, idx2), (2 * _PS, idx3)):
        for j in range(_NCHUNK):
            for l in range(_CHUNK // _D):
                sl = pl.ds(l * _D, _D)
                idx[j, sl] = idx[j, sl] + off

    copies = []
    for j in range(_NCHUNK):
        sl = pl.ds(j * _CHUNK, _CHUNK)
        copies.append(pltpu.async_copy(p_hbm.at[idx1.at[j]], g1.at[sl], sem))
        copies.append(pltpu.async_copy(p_hbm.at[idx2.at[j]], g2.at[sl], sem))
        copies.append(pltpu.async_copy(p_hbm.at[idx3.at[j]], g3.at[sl], sem))
    for cp in copies:
        cp.wait()

    def group_body(g, carry):
        sl = pl.ds(g * _D, _D)
        out_v[sl] = g1[sl] + g2[sl] + g3[sl]
        return carry

    lax.fori_loop(0, _G, group_body, 0)

    pltpu.sync_copy(out_v, out_hbm.at[wid])


def kernel(f1, f2, f3, emb1, emb2, W, b):
    f1 = f1.astype(jnp.int32).reshape(_NW, _NCHUNK, _CHUNK)
    f2 = f2.astype(jnp.int32).reshape(_NW, _NCHUNK, _CHUNK)
    f3 = f3.astype(jnp.int32).reshape(_NW, _NCHUNK, _CHUNK)
    wm = W.astype(jnp.float32).reshape(3, _D)
    bm = b.astype(jnp.float32).reshape(1, 1)
    p = _proj(emb1.T, emb2.T, wm, bm)
    out = _gather_add_sc(f1, f2, f3, p)
    return out.reshape(_B, 1)


# speedup vs baseline: 2.0104x; 2.0104x over previous
import functools

import jax
import jax.numpy as jnp
from jax import lax
from jax.experimental import pallas as pl
from jax.experimental.pallas import tpu as pltpu
from jax.experimental.pallas import tpu_sc as plsc

_B = 16384
_V = 1000000
_D = 16
_NC = 2
_NS = 16
_NW = _NC * _NS
_BPW = _B // _NW
_CHUNK = 128
_NCHUNK = _BPW // _CHUNK
_G = _BPW // _D

_BC = 16384
_NBLK = -(-_V // _BC)
_PS = _NBLK * _BC

_mesh = plsc.VectorSubcoreMesh(core_axis_name="c", subcore_axis_name="s")


def _proj_body(e1_ref, e2_ref, wm_ref, b_ref, p0_ref, p1_ref, p2_ref):
    e1 = e1_ref[...]
    e2 = e2_ref[...]
    wm = wm_ref[...]
    p01 = jax.lax.dot_general(
        wm[0:2], e1, (((1,), (0,)), ((), ())),
        preferred_element_type=jnp.float32)
    p2 = jax.lax.dot_general(
        wm[2:3], e2, (((1,), (0,)), ((), ())),
        preferred_element_type=jnp.float32)
    p0_ref[...] = p01[0]
    p1_ref[...] = p01[1]
    p2_ref[...] = p2[0] + b_ref[0, 0]


_proj = pl.pallas_call(
    _proj_body,
    grid=(_NBLK,),
    in_specs=[
        pl.BlockSpec((_D, _BC), lambda i: (0, i)),
        pl.BlockSpec((_D, _BC), lambda i: (0, i)),
        pl.BlockSpec((3, _D), lambda i: (0, 0)),
        pl.BlockSpec(memory_space=pltpu.SMEM),
    ],
    out_specs=[
        pl.BlockSpec((_BC,), lambda i: (i,)),
        pl.BlockSpec((_BC,), lambda i: (i,)),
        pl.BlockSpec((_BC,), lambda i: (i,)),
    ],
    out_shape=[
        jax.ShapeDtypeStruct((_PS,), jnp.float32),
        jax.ShapeDtypeStruct((_PS,), jnp.float32),
        jax.ShapeDtypeStruct((_PS,), jnp.float32),
    ],
)


@functools.partial(
    pl.kernel,
    out_type=jax.ShapeDtypeStruct((_NW, _BPW), jnp.float32),
    mesh=_mesh,
    scratch_types=[
        pltpu.VMEM((_NCHUNK, _CHUNK), jnp.int32),
        pltpu.VMEM((_NCHUNK, _CHUNK), jnp.int32),
        pltpu.VMEM((_NCHUNK, _CHUNK), jnp.int32),
        pltpu.VMEM((_BPW,), jnp.float32),
        pltpu.VMEM((_BPW,), jnp.float32),
        pltpu.VMEM((_BPW,), jnp.float32),
        pltpu.VMEM((_BPW,), jnp.float32),
        pltpu.SemaphoreType.DMA,
    ],
    compiler_params=pltpu.CompilerParams(
        needs_layout_passes=False, use_tc_tiling_on_sc=False),
)
def _gather_add_sc(f1_hbm, f2_hbm, f3_hbm, p0_hbm, p1_hbm, p2_hbm, out_hbm,
                   idx1, idx2, idx3, g1, g2, g3, out_v, sem):
    wid = lax.axis_index("s") * _NC + lax.axis_index("c")

    pltpu.sync_copy(f1_hbm.at[wid], idx1)
    pltpu.sync_copy(f2_hbm.at[wid], idx2)
    pltpu.sync_copy(f3_hbm.at[wid], idx3)

    copies = []
    for j in range(_NCHUNK):
        sl = pl.ds(j * _CHUNK, _CHUNK)
        copies.append(pltpu.async_copy(p0_hbm.at[idx1.at[j]], g1.at[sl], sem))
        copies.append(pltpu.async_copy(p1_hbm.at[idx2.at[j]], g2.at[sl], sem))
        copies.append(pltpu.async_copy(p2_hbm.at[idx3.at[j]], g3.at[sl], sem))
    for cp in copies:
        cp.wait()

    def group_body(g, carry):
        sl = pl.ds(g * _D, _D)
        out_v[sl] = g1[sl] + g2[sl] + g3[sl]
        return carry

    lax.fori_loop(0, _G, group_body, 0)

    pltpu.sync_copy(out_v, out_hbm.at[wid])


def kernel(f1, f2, f3, emb1, emb2, W, b):
    f1 = f1.astype(jnp.int32).reshape(_NW, _NCHUNK, _CHUNK)
    f2 = f2.astype(jnp.int32).reshape(_NW, _NCHUNK, _CHUNK)
    f3 = f3.astype(jnp.int32).reshape(_NW, _NCHUNK, _CHUNK)
    wm = W.astype(jnp.float32).reshape(3, _D)
    bm = b.astype(jnp.float32).reshape(1, 1)
    p0, p1, p2 = _proj(emb1.T, emb2.T, wm, bm)
    out = _gather_add_sc(f1, f2, f3, p0, p1, p2)
    return out.reshape(_B, 1)


# speedup vs baseline: 2.4929x; 1.2400x over previous
import functools

import jax
import jax.numpy as jnp
from jax import lax
from jax.experimental import pallas as pl
from jax.experimental.pallas import tpu as pltpu
from jax.experimental.pallas import tpu_sc as plsc

_B = 16384
_V = 1000000
_D = 16
_NC = 2
_NS = 16
_NW = _NC * _NS
_BPW = _B // _NW
_CHUNK = 128
_NCHUNK = _BPW // _CHUNK
_G = _BPW // _D

_BC = 32768
_NBLK = -(-_V // _BC)
_PS = _NBLK * _BC

_mesh = plsc.VectorSubcoreMesh(core_axis_name="c", subcore_axis_name="s")


def _proj_body(e1_ref, e2_ref, wm_ref, b_ref, p0_ref, p1_ref, p2_ref):
    e1 = e1_ref[...]
    e2 = e2_ref[...]
    wm = wm_ref[...]
    p01 = jax.lax.dot_general(
        wm[0:2], e1, (((1,), (0,)), ((), ())),
        preferred_element_type=jnp.float32)
    p2 = jax.lax.dot_general(
        wm[2:3], e2, (((1,), (0,)), ((), ())),
        preferred_element_type=jnp.float32)
    p0_ref[...] = p01[0]
    p1_ref[...] = p01[1]
    p2_ref[...] = p2[0] + b_ref[0, 0]


_proj = pl.pallas_call(
    _proj_body,
    grid=(_NBLK,),
    in_specs=[
        pl.BlockSpec((_D, _BC), lambda i: (0, i)),
        pl.BlockSpec((_D, _BC), lambda i: (0, i)),
        pl.BlockSpec((3, _D), lambda i: (0, 0)),
        pl.BlockSpec(memory_space=pltpu.SMEM),
    ],
    out_specs=[
        pl.BlockSpec((_BC,), lambda i: (i,)),
        pl.BlockSpec((_BC,), lambda i: (i,)),
        pl.BlockSpec((_BC,), lambda i: (i,)),
    ],
    out_shape=[
        jax.ShapeDtypeStruct((_PS,), jnp.float32),
        jax.ShapeDtypeStruct((_PS,), jnp.float32),
        jax.ShapeDtypeStruct((_PS,), jnp.float32),
    ],
)


@functools.partial(
    pl.kernel,
    out_type=jax.ShapeDtypeStruct((_NW, _BPW), jnp.float32),
    mesh=_mesh,
    scratch_types=[
        pltpu.VMEM((_NCHUNK, _CHUNK), jnp.int32),
        pltpu.VMEM((_NCHUNK, _CHUNK), jnp.int32),
        pltpu.VMEM((_NCHUNK, _CHUNK), jnp.int32),
        pltpu.VMEM((_BPW,), jnp.float32),
        pltpu.VMEM((_BPW,), jnp.float32),
        pltpu.VMEM((_BPW,), jnp.float32),
        pltpu.VMEM((_BPW,), jnp.float32),
        pltpu.SemaphoreType.DMA,
    ],
    compiler_params=pltpu.CompilerParams(
        needs_layout_passes=False, use_tc_tiling_on_sc=False),
)
def _gather_add_sc(f1_hbm, f2_hbm, f3_hbm, p0_hbm, p1_hbm, p2_hbm, out_hbm,
                   idx1, idx2, idx3, g1, g2, g3, out_v, sem):
    wid = lax.axis_index("s") * _NC + lax.axis_index("c")

    pltpu.sync_copy(f1_hbm.at[wid], idx1)
    pltpu.sync_copy(f2_hbm.at[wid], idx2)
    pltpu.sync_copy(f3_hbm.at[wid], idx3)

    copies = []
    for j in range(_NCHUNK):
        sl = pl.ds(j * _CHUNK, _CHUNK)
        copies.append(pltpu.async_copy(p0_hbm.at[idx1.at[j]], g1.at[sl], sem))
        copies.append(pltpu.async_copy(p1_hbm.at[idx2.at[j]], g2.at[sl], sem))
        copies.append(pltpu.async_copy(p2_hbm.at[idx3.at[j]], g3.at[sl], sem))
    for cp in copies:
        cp.wait()

    def group_body(g, carry):
        sl = pl.ds(g * _D, _D)
        out_v[sl] = g1[sl] + g2[sl] + g3[sl]
        return carry

    lax.fori_loop(0, _G, group_body, 0)

    pltpu.sync_copy(out_v, out_hbm.at[wid])


def kernel(f1, f2, f3, emb1, emb2, W, b):
    f1 = f1.astype(jnp.int32).reshape(_NW, _NCHUNK, _CHUNK)
    f2 = f2.astype(jnp.int32).reshape(_NW, _NCHUNK, _CHUNK)
    f3 = f3.astype(jnp.int32).reshape(_NW, _NCHUNK, _CHUNK)
    wm = W.astype(jnp.float32).reshape(3, _D)
    bm = b.astype(jnp.float32).reshape(1, 1)
    p0, p1, p2 = _proj(emb1.T, emb2.T, wm, bm)
    out = _gather_add_sc(f1, f2, f3, p0, p1, p2)
    return out.reshape(_B, 1)


# speedup vs baseline: 2.6751x; 1.0731x over previous
import functools

import jax
import jax.numpy as jnp
from jax import lax
from jax.experimental import pallas as pl
from jax.experimental.pallas import tpu as pltpu
from jax.experimental.pallas import tpu_sc as plsc

_B = 16384
_V = 1000000
_D = 16
_NC = 2
_NS = 16
_NW = _NC * _NS
_BPW = _B // _NW
_CHUNK = 128
_NCHUNK = _BPW // _CHUNK
_G = _BPW // _D

_BC = 65536
_NBLK = -(-_V // _BC)
_PS = _NBLK * _BC

_mesh = plsc.VectorSubcoreMesh(core_axis_name="c", subcore_axis_name="s")


def _proj_body(e1_ref, e2_ref, wm_ref, b_ref, p0_ref, p1_ref, p2_ref):
    e1 = e1_ref[...]
    e2 = e2_ref[...]
    wm = wm_ref[...]
    p01 = jax.lax.dot_general(
        wm[0:2], e1, (((1,), (0,)), ((), ())),
        preferred_element_type=jnp.float32)
    p2 = jax.lax.dot_general(
        wm[2:3], e2, (((1,), (0,)), ((), ())),
        preferred_element_type=jnp.float32)
    p0_ref[...] = p01[0]
    p1_ref[...] = p01[1]
    p2_ref[...] = p2[0] + b_ref[0, 0]


_proj = pl.pallas_call(
    _proj_body,
    grid=(_NBLK,),
    in_specs=[
        pl.BlockSpec((_D, _BC), lambda i: (0, i)),
        pl.BlockSpec((_D, _BC), lambda i: (0, i)),
        pl.BlockSpec((3, _D), lambda i: (0, 0)),
        pl.BlockSpec(memory_space=pltpu.SMEM),
    ],
    out_specs=[
        pl.BlockSpec((_BC,), lambda i: (i,)),
        pl.BlockSpec((_BC,), lambda i: (i,)),
        pl.BlockSpec((_BC,), lambda i: (i,)),
    ],
    out_shape=[
        jax.ShapeDtypeStruct((_PS,), jnp.float32),
        jax.ShapeDtypeStruct((_PS,), jnp.float32),
        jax.ShapeDtypeStruct((_PS,), jnp.float32),
    ],
)


@functools.partial(
    pl.kernel,
    out_type=jax.ShapeDtypeStruct((_NW, _BPW), jnp.float32),
    mesh=_mesh,
    scratch_types=[
        pltpu.VMEM((_NCHUNK, _CHUNK), jnp.int32),
        pltpu.VMEM((_NCHUNK, _CHUNK), jnp.int32),
        pltpu.VMEM((_NCHUNK, _CHUNK), jnp.int32),
        pltpu.VMEM((_BPW,), jnp.float32),
        pltpu.VMEM((_BPW,), jnp.float32),
        pltpu.VMEM((_BPW,), jnp.float32),
        pltpu.VMEM((_BPW,), jnp.float32),
        pltpu.SemaphoreType.DMA,
    ],
    compiler_params=pltpu.CompilerParams(
        needs_layout_passes=False, use_tc_tiling_on_sc=False),
)
def _gather_add_sc(f1_hbm, f2_hbm, f3_hbm, p0_hbm, p1_hbm, p2_hbm, out_hbm,
                   idx1, idx2, idx3, g1, g2, g3, out_v, sem):
    wid = lax.axis_index("s") * _NC + lax.axis_index("c")

    pltpu.sync_copy(f1_hbm.at[wid], idx1)
    pltpu.sync_copy(f2_hbm.at[wid], idx2)
    pltpu.sync_copy(f3_hbm.at[wid], idx3)

    copies = []
    for j in range(_NCHUNK):
        sl = pl.ds(j * _CHUNK, _CHUNK)
        copies.append(pltpu.async_copy(p0_hbm.at[idx1.at[j]], g1.at[sl], sem))
        copies.append(pltpu.async_copy(p1_hbm.at[idx2.at[j]], g2.at[sl], sem))
        copies.append(pltpu.async_copy(p2_hbm.at[idx3.at[j]], g3.at[sl], sem))
    for cp in copies:
        cp.wait()

    def group_body(g, carry):
        sl = pl.ds(g * _D, _D)
        out_v[sl] = g1[sl] + g2[sl] + g3[sl]
        return carry

    lax.fori_loop(0, _G, group_body, 0)

    pltpu.sync_copy(out_v, out_hbm.at[wid])


def kernel(f1, f2, f3, emb1, emb2, W, b):
    f1 = f1.astype(jnp.int32).reshape(_NW, _NCHUNK, _CHUNK)
    f2 = f2.astype(jnp.int32).reshape(_NW, _NCHUNK, _CHUNK)
    f3 = f3.astype(jnp.int32).reshape(_NW, _NCHUNK, _CHUNK)
    wm = W.astype(jnp.float32).reshape(3, _D)
    bm = b.astype(jnp.float32).reshape(1, 1)
    p0, p1, p2 = _proj(emb1.T, emb2.T, wm, bm)
    out = _gather_add_sc(f1, f2, f3, p0, p1, p2)
    return out.reshape(_B, 1)
